# Initial kernel scaffold; baseline (speedup 1.0000x reference)
import jax
import jax.numpy as jnp
from jax.experimental import pallas as pl


def kernel(x, z, latents, Wq, Wk, Wv, Wo):
    raise NotImplementedError("write your pallas kernel here")


# speedup vs baseline: 1.4658x; 1.4658x over previous
import jax
import jax.numpy as jnp
import numpy as np
from jax import lax
from jax.experimental import pallas as pl
from jax.experimental.pallas import tpu as pltpu
from jax.experimental.pallas import tpu_sc as plsc

E = 256
H = 8
HD = E // H
P0, P1 = 32, 32
G = P0 * P1
M = 4
N = 8192
NTOK = M * N
NC, NS = 2, 16
NW = NC * NS
CH = 128
RPS = NTOK // NC
RWS = 2 * G
F32 = jnp.float32
SCALE = np.float32(1.0 / np.sqrt(HD))


def _cells_body(x0_ref, x1_ref, cell_ref, sidx_ref):
    sp = np.float32(1.0) / np.float32(P0 - 1)
    half = sp / np.float32(2.0)
    i0 = jnp.clip(jnp.floor((x0_ref[...] + half) / sp), 0.0, np.float32(P0 - 1))
    i1 = jnp.clip(jnp.floor((x1_ref[...] + half) / sp), 0.0, np.float32(P1 - 1))
    cell = (i0 * np.float32(P1) + i1).astype(jnp.int32)
    cell_ref[...] = cell
    b = lax.broadcasted_iota(jnp.int32, (M, N), 0)
    sidx_ref[...] = cell + (b % 2) * G


def _lat_body(latfT_ref, wq_ref, wk_ref, wv_ref, s_ref, e816_ref,
              qlat_ref, vgT_ref, ps16T_ref):
    latfT = latfT_ref[...]
    q = lax.dot_general(latfT, wq_ref[...], (((0,), (0,)), ((), ())),
                        preferred_element_type=F32)
    kg = lax.dot_general(latfT, wk_ref[...], (((0,), (0,)), ((), ())),
                         preferred_element_type=F32)
    vgT_ref[...] = lax.dot_general(wv_ref[...], latfT, (((0,), (0,)), ((), ())),
                                   preferred_element_type=F32)
    qlat_ref[...] = q
    l8 = jnp.dot(q * kg, s_ref[...], preferred_element_type=F32) * SCALE
    psT = jnp.exp(l8).T
    ps16T_ref[...] = lax.dot_general(e816_ref[...], psT, (((0,), (0,)), ((), ())),
                                     preferred_element_type=F32)


def _fused_w_body(z_ref, qt_ref, wk_ref, wv_ref, s_ref, e816_ref,
                  wT_ref, p16T_ref):
    zb = z_ref[...]
    k = jnp.dot(zb, wk_ref[...], preferred_element_type=F32)
    l8 = jnp.dot(qt_ref[...] * k, s_ref[...], preferred_element_type=F32) * SCALE
    pT = jnp.exp(l8).T
    vT = lax.dot_general(wv_ref[...], zb, (((0,), (1,)), ((), ())),
                         preferred_element_type=F32)
    prepT = jnp.dot(s_ref[...], pT, preferred_element_type=F32)
    wT_ref[...] = prepT * vT
    p16T_ref[...] = lax.dot_general(e816_ref[...], pT, (((0,), (0,)), ((), ())),
                                    preferred_element_type=F32)


def _final_body(numT_ref, denT_ref, ps16T_ref, vgT_ref, wo_ref, st16_ref,
                out_ref):
    ps_repT = lax.dot_general(st16_ref[...], ps16T_ref[...],
                              (((0,), (0,)), ((), ())),
                              preferred_element_type=F32)
    numerT = numT_ref[...] + ps_repT * vgT_ref[...]
    den_repT = lax.dot_general(st16_ref[...], denT_ref[...] + ps16T_ref[...],
                               (((0,), (0,)), ((), ())),
                               preferred_element_type=F32)
    attnT = numerT / den_repT
    out_ref[...] = lax.dot_general(attnT, wo_ref[...], (((0,), (0,)), ((), ())),
                                   preferred_element_type=F32)


def _sc_gather_body(qlat_hbm, cell_hbm, out_hbm, idx_v, rows_v, sem):
    c = lax.axis_index("c")
    s = lax.axis_index("s")
    wid = s * NC + c
    per_w = NTOK // NW

    def chunk(i, carry):
        base = wid * per_w + i * CH
        pltpu.sync_copy(cell_hbm.at[pl.ds(base, CH)], idx_v)
        pltpu.async_copy(qlat_hbm.at[idx_v], rows_v, sem).wait()
        pltpu.sync_copy(rows_v, out_hbm.at[pl.ds(base, CH)])
        return carry

    lax.fori_loop(0, per_w // CH, chunk, 0)


def _sc_scatter_body(wT_hbm, p16T_hbm, sidx_hbm, zrow_hbm,
                     numT_hbm, denT_hbm,
                     cell_v, w_v, acc, pbuf, sh_p):
    c = lax.axis_index("c")
    s = lax.axis_index("s")
    iota16 = lax.iota(jnp.int32, 16)

    def accum_chunk(base):
        pltpu.sync_copy(sidx_hbm.at[pl.ds(base, CH)], cell_v)

        def tok(t, carry):
            tsplat = jnp.full((16,), t, jnp.int32)
            row = plsc.load_gather(cell_v, [tsplat])
            vals = plsc.load_gather(w_v, [iota16, tsplat])
            plsc.addupdate_scatter(acc, [iota16, row], vals)
            return carry

        lax.fori_loop(0, CH, tok, 0)

    pltpu.sync_copy(zrow_hbm, acc)

    def chunk_n(i, carry):
        base = c * RPS + i * CH
        pltpu.sync_copy(wT_hbm.at[pl.ds(s * 16, 16), pl.ds(base, CH)], w_v)
        accum_chunk(base)
        return carry

    lax.fori_loop(0, RPS // CH, chunk_n, 0)
    pltpu.sync_copy(acc, numT_hbm.at[pl.ds(s * 16, 16), pl.ds(c * RWS, RWS)])

    pltpu.sync_copy(zrow_hbm, acc)
    per_tile = RPS // NS

    def chunk_d(i, carry):
        base = c * RPS + s * per_tile + i * CH
        pltpu.sync_copy(p16T_hbm.at[:, pl.ds(base, CH)], w_v)
        accum_chunk(base)
        return carry

    lax.fori_loop(0, per_tile // CH, chunk_d, 0)
    pltpu.sync_copy(acc, sh_p.at[s])
    plsc.subcore_barrier()

    cs = RWS // NS
    pltpu.sync_copy(sh_p.at[:, :, pl.ds(s * cs, cs)], pbuf)

    def comb(i, carry):
        h = i // (cs // 16)
        c0 = (i % (cs // 16)) * 16

        def part(j, a):
            return a + pbuf[j, h, pl.ds(c0, 16)]

        w_v[h, pl.ds(c0, 16)] = lax.fori_loop(0, NS, part, jnp.zeros((16,), F32))
        return carry

    lax.fori_loop(0, 16 * (cs // 16), comb, 0)
    pltpu.sync_copy(w_v.at[:, pl.ds(0, cs)],
                    denT_hbm.at[:, pl.ds(c * RWS + s * cs, cs)])


def kernel(x, z, latents, Wq, Wk, Wv, Wo):
    hid = np.arange(E) // HD
    s_mat = jnp.asarray((hid[:, None] == np.arange(H)[None, :]).astype(np.float32))
    e816 = jnp.asarray(np.eye(H, 16, dtype=np.float32))
    st16 = jnp.asarray((hid[None, :] == np.arange(16)[:, None]).astype(np.float32))

    cell, sidx = pl.pallas_call(
        _cells_body,
        out_shape=(jax.ShapeDtypeStruct((M, N), jnp.int32),
                   jax.ShapeDtypeStruct((M, N), jnp.int32)),
    )(x[..., 0], x[..., 1])

    latfT = latents.reshape(G, E).T
    qlat, vgT, ps16T = pl.pallas_call(
        _lat_body,
        out_shape=(jax.ShapeDtypeStruct((G, E), F32),
                   jax.ShapeDtypeStruct((E, G), F32),
                   jax.ShapeDtypeStruct((16, G), F32)),
    )(latfT, Wq, Wk, Wv, s_mat, e816)

    mesh = plsc.VectorSubcoreMesh(core_axis_name="c", subcore_axis_name="s",
                                  num_cores=NC, num_subcores=NS)

    qt = pl.kernel(
        _sc_gather_body,
        out_type=jax.ShapeDtypeStruct((NTOK, E), F32),
        mesh=mesh,
        scratch_types=[
            pltpu.VMEM((CH,), jnp.int32),
            pltpu.VMEM((CH, E), F32),
            pltpu.SemaphoreType.DMA,
        ],
    )(qlat, cell.reshape(NTOK))

    BT = 1024
    wT, p16T = pl.pallas_call(
        _fused_w_body,
        grid=(NTOK // BT,),
        in_specs=[
            pl.BlockSpec((BT, E), lambda i: (i, 0)),
            pl.BlockSpec((BT, E), lambda i: (i, 0)),
            pl.BlockSpec((E, E), lambda i: (0, 0)),
            pl.BlockSpec((E, E), lambda i: (0, 0)),
            pl.BlockSpec((E, H), lambda i: (0, 0)),
            pl.BlockSpec((H, 16), lambda i: (0, 0)),
        ],
        out_specs=(pl.BlockSpec((E, BT), lambda i: (0, i)),
                   pl.BlockSpec((16, BT), lambda i: (0, i))),
        out_shape=(jax.ShapeDtypeStruct((E, NTOK), F32),
                   jax.ShapeDtypeStruct((16, NTOK), F32)),
    )(z.reshape(NTOK, E), qt, Wk, Wv, s_mat, e816)

    zrow = jnp.zeros((16, RWS), F32)
    numT, denT = pl.kernel(
        _sc_scatter_body,
        out_type=(jax.ShapeDtypeStruct((E, M * G), F32),
                  jax.ShapeDtypeStruct((16, M * G), F32)),
        mesh=mesh,
        compiler_params=pltpu.CompilerParams(needs_layout_passes=False),
        scratch_types=[
            pltpu.VMEM((CH,), jnp.int32),
            pltpu.VMEM((16, CH), F32),
            pltpu.VMEM((16, RWS), F32),
            pltpu.VMEM((NS, 16, RWS // NS), F32),
            pltpu.VMEM_SHARED((NS, 16, RWS), F32),
        ],
    )(wT, p16T, sidx.reshape(NTOK), zrow)

    out = pl.pallas_call(
        _final_body,
        grid=(M,),
        in_specs=[
            pl.BlockSpec((E, G), lambda i: (0, i)),
            pl.BlockSpec((16, G), lambda i: (0, i)),
            pl.BlockSpec((16, G), lambda i: (0, 0)),
            pl.BlockSpec((E, G), lambda i: (0, 0)),
            pl.BlockSpec((E, E), lambda i: (0, 0)),
            pl.BlockSpec((16, E), lambda i: (0, 0)),
        ],
        out_specs=pl.BlockSpec((G, E), lambda i: (i, 0)),
        out_shape=jax.ShapeDtypeStruct((M * G, E), F32),
    )(numT, denT, ps16T, vgT, Wo, st16)

    z_grid = out.reshape(M, P0, P1, E)

    axes = [jnp.linspace(r[0], r[1], p, dtype=F32)
            for r, p in zip(((0.0, 1.0), (0.0, 1.0)), (P0, P1))]
    grid_pts = jnp.stack(jnp.meshgrid(*axes, indexing="ij"), axis=-1)
    x_grid = jnp.broadcast_to(grid_pts[None], (M, P0, P1, 2))
    return x_grid, z_grid


# speedup vs baseline: 4.4708x; 3.0500x over previous
import jax
import jax.numpy as jnp
import numpy as np
from jax import lax
from jax.experimental import pallas as pl
from jax.experimental.pallas import tpu as pltpu
from jax.experimental.pallas import tpu_sc as plsc

E = 256
H = 8
HD = E // H
P0, P1 = 32, 32
G = P0 * P1
M = 4
N = 8192
NTOK = M * N
NC, NS = 2, 16
NW = NC * NS
CH = 128
RPS = NTOK // NC
RWS = 2 * G
F32 = jnp.float32
SCALE = np.float32(1.0 / np.sqrt(HD))


def _cells_body(x0_ref, x1_ref, cell_ref, sidx_ref):
    sp = np.float32(1.0) / np.float32(P0 - 1)
    half = sp / np.float32(2.0)
    i0 = jnp.clip(jnp.floor((x0_ref[...] + half) / sp), 0.0, np.float32(P0 - 1))
    i1 = jnp.clip(jnp.floor((x1_ref[...] + half) / sp), 0.0, np.float32(P1 - 1))
    cell = (i0 * np.float32(P1) + i1).astype(jnp.int32)
    cell_ref[...] = cell
    b = lax.broadcasted_iota(jnp.int32, (M, N), 0)
    sidx_ref[...] = cell + (b % 2) * G


def _lat_body(latfT_ref, wq_ref, wk_ref, wv_ref, s_ref, e816_ref,
              qlat_ref, vgT_ref, ps16T_ref):
    latfT = latfT_ref[...]
    q = lax.dot_general(latfT, wq_ref[...], (((0,), (0,)), ((), ())),
                        preferred_element_type=F32)
    kg = lax.dot_general(latfT, wk_ref[...], (((0,), (0,)), ((), ())),
                         preferred_element_type=F32)
    vgT_ref[...] = lax.dot_general(wv_ref[...], latfT, (((0,), (0,)), ((), ())),
                                   preferred_element_type=F32)
    qlat_ref[...] = q
    l8 = jnp.dot(q * kg, s_ref[...], preferred_element_type=F32) * SCALE
    psT = jnp.exp(l8).T
    ps16T_ref[...] = lax.dot_general(e816_ref[...], psT, (((0,), (0,)), ((), ())),
                                     preferred_element_type=F32)


def _fused_w_body(z_ref, qt_ref, wk_ref, wv_ref, s_ref, e816_ref,
                  wT_ref, p16T_ref):
    zb = z_ref[...]
    k = jnp.dot(zb, wk_ref[...], preferred_element_type=F32)
    l8 = jnp.dot(qt_ref[...] * k, s_ref[...], preferred_element_type=F32) * SCALE
    pT = jnp.exp(l8).T
    vT = lax.dot_general(wv_ref[...], zb, (((0,), (1,)), ((), ())),
                         preferred_element_type=F32)
    prepT = jnp.dot(s_ref[...], pT, preferred_element_type=F32)
    wT_ref[...] = prepT * vT
    p16T_ref[...] = lax.dot_general(e816_ref[...], pT, (((0,), (0,)), ((), ())),
                                    preferred_element_type=F32)


def _final_body(numT_ref, denT_ref, ps16T_ref, vgT_ref, wo_ref, st16_ref,
                out_ref):
    ps_repT = lax.dot_general(st16_ref[...], ps16T_ref[...],
                              (((0,), (0,)), ((), ())),
                              preferred_element_type=F32)
    numerT = numT_ref[...] + ps_repT * vgT_ref[...]
    den_repT = lax.dot_general(st16_ref[...], denT_ref[...] + ps16T_ref[...],
                               (((0,), (0,)), ((), ())),
                               preferred_element_type=F32)
    attnT = numerT / den_repT
    out_ref[...] = lax.dot_general(attnT, wo_ref[...], (((0,), (0,)), ((), ())),
                                   preferred_element_type=F32)


def _sc_gather_body(qlat_hbm, cell_hbm, out_hbm, idx_v, rows_v, sem):
    c = lax.axis_index("c")
    s = lax.axis_index("s")
    wid = s * NC + c
    per_w = NTOK // NW

    def chunk(i, carry):
        base = wid * per_w + i * CH
        pltpu.sync_copy(cell_hbm.at[pl.ds(base, CH)], idx_v)
        pltpu.async_copy(qlat_hbm.at[idx_v], rows_v, sem).wait()
        pltpu.sync_copy(rows_v, out_hbm.at[pl.ds(base, CH)])
        return carry

    lax.fori_loop(0, per_w // CH, chunk, 0)


def _sc_scatter_body(wT_hbm, p16T_hbm, sidx_hbm, zrow_hbm,
                     numT_hbm, denT_hbm,
                     cell_v0, cell_v1, w_v0, w_v1, acc, pbuf, sh_p,
                     sem0, sem1):
    c = lax.axis_index("c")
    s = lax.axis_index("s")
    bufs = ((cell_v0, w_v0, sem0), (cell_v1, w_v1, sem1))

    def accumulate(cell_v, w_v):
        def tblk(t, carry):
            rows = cell_v[pl.ds(t * 16, 16)]
            for j in range(16):
                vals = w_v[j, pl.ds(t * 16, 16)]
                plsc.addupdate_scatter(acc, [jnp.full((16,), j, jnp.int32), rows],
                                       vals)
            return carry

        lax.fori_loop(0, CH // 16, tblk, 0)

    def start(i, src_hbm, row0, nrows, tok0, b):
        cell_v, w_v, sem = bufs[b]
        base = tok0 + i * CH
        cp1 = pltpu.async_copy(sidx_hbm.at[pl.ds(base, CH)], cell_v, sem)
        cp2 = pltpu.async_copy(src_hbm.at[pl.ds(row0, nrows), pl.ds(base, CH)],
                               w_v.at[pl.ds(0, nrows)], sem)
        return cp1, cp2

    def run_pass(src_hbm, row0, nrows, tok0, nchunks):
        start(0, src_hbm, row0, nrows, tok0, 0)

        def pair(i, carry):
            for b in range(2):
                ci = i * 2 + b
                cell_v, w_v, sem = bufs[b]
                pltpu.make_async_copy(sidx_hbm.at[pl.ds(0, CH)], cell_v, sem).wait()
                pltpu.make_async_copy(
                    src_hbm.at[pl.ds(row0, nrows), pl.ds(0, CH)],
                    w_v.at[pl.ds(0, nrows)], sem).wait()

                @pl.when(ci + 1 < nchunks)
                def _():
                    start(ci + 1, src_hbm, row0, nrows, tok0, 1 - b)

                accumulate(cell_v, w_v)
            return carry

        lax.fori_loop(0, nchunks // 2, pair, 0)

    pltpu.sync_copy(zrow_hbm, acc)
    run_pass(wT_hbm, s * 16, 16, c * RPS, RPS // CH)
    pltpu.sync_copy(acc, numT_hbm.at[pl.ds(s * 16, 16), pl.ds(c * RWS, RWS)])

    pltpu.sync_copy(zrow_hbm, acc)
    per_tile = RPS // NS
    run_pass(p16T_hbm, 0, 16, c * RPS + s * per_tile, per_tile // CH)
    pltpu.sync_copy(acc, sh_p.at[s])
    plsc.subcore_barrier()

    cs = RWS // NS
    pltpu.sync_copy(sh_p.at[:, :, pl.ds(s * cs, cs)], pbuf)

    def comb(i, carry):
        h = i // (cs // 16)
        c0 = (i % (cs // 16)) * 16

        def part(j, a):
            return a + pbuf[j, h, pl.ds(c0, 16)]

        w_v0[h, pl.ds(c0, 16)] = lax.fori_loop(0, NS, part, jnp.zeros((16,), F32))
        return carry

    lax.fori_loop(0, 16 * (cs // 16), comb, 0)
    pltpu.sync_copy(w_v0.at[:, pl.ds(0, cs)],
                    denT_hbm.at[:, pl.ds(c * RWS + s * cs, cs)])


def kernel(x, z, latents, Wq, Wk, Wv, Wo):
    hid = np.arange(E) // HD
    s_mat = jnp.asarray((hid[:, None] == np.arange(H)[None, :]).astype(np.float32))
    e816 = jnp.asarray(np.eye(H, 16, dtype=np.float32))
    st16 = jnp.asarray((hid[None, :] == np.arange(16)[:, None]).astype(np.float32))

    cell, sidx = pl.pallas_call(
        _cells_body,
        out_shape=(jax.ShapeDtypeStruct((M, N), jnp.int32),
                   jax.ShapeDtypeStruct((M, N), jnp.int32)),
    )(x[..., 0], x[..., 1])

    latfT = latents.reshape(G, E).T
    qlat, vgT, ps16T = pl.pallas_call(
        _lat_body,
        out_shape=(jax.ShapeDtypeStruct((G, E), F32),
                   jax.ShapeDtypeStruct((E, G), F32),
                   jax.ShapeDtypeStruct((16, G), F32)),
    )(latfT, Wq, Wk, Wv, s_mat, e816)

    mesh = plsc.VectorSubcoreMesh(core_axis_name="c", subcore_axis_name="s",
                                  num_cores=NC, num_subcores=NS)

    qt = pl.kernel(
        _sc_gather_body,
        out_type=jax.ShapeDtypeStruct((NTOK, E), F32),
        mesh=mesh,
        scratch_types=[
            pltpu.VMEM((CH,), jnp.int32),
            pltpu.VMEM((CH, E), F32),
            pltpu.SemaphoreType.DMA,
        ],
    )(qlat, cell.reshape(NTOK))

    BT = 1024
    wT, p16T = pl.pallas_call(
        _fused_w_body,
        grid=(NTOK // BT,),
        in_specs=[
            pl.BlockSpec((BT, E), lambda i: (i, 0)),
            pl.BlockSpec((BT, E), lambda i: (i, 0)),
            pl.BlockSpec((E, E), lambda i: (0, 0)),
            pl.BlockSpec((E, E), lambda i: (0, 0)),
            pl.BlockSpec((E, H), lambda i: (0, 0)),
            pl.BlockSpec((H, 16), lambda i: (0, 0)),
        ],
        out_specs=(pl.BlockSpec((E, BT), lambda i: (0, i)),
                   pl.BlockSpec((16, BT), lambda i: (0, i))),
        out_shape=(jax.ShapeDtypeStruct((E, NTOK), F32),
                   jax.ShapeDtypeStruct((16, NTOK), F32)),
    )(z.reshape(NTOK, E), qt, Wk, Wv, s_mat, e816)

    zrow = jnp.zeros((16, RWS), F32)
    numT, denT = pl.kernel(
        _sc_scatter_body,
        out_type=(jax.ShapeDtypeStruct((E, M * G), F32),
                  jax.ShapeDtypeStruct((16, M * G), F32)),
        mesh=mesh,
        compiler_params=pltpu.CompilerParams(needs_layout_passes=False),
        scratch_types=[
            pltpu.VMEM((CH,), jnp.int32),
            pltpu.VMEM((CH,), jnp.int32),
            pltpu.VMEM((16, CH), F32),
            pltpu.VMEM((16, CH), F32),
            pltpu.VMEM((16, RWS), F32),
            pltpu.VMEM((NS, 16, RWS // NS), F32),
            pltpu.VMEM_SHARED((NS, 16, RWS), F32),
            pltpu.SemaphoreType.DMA,
            pltpu.SemaphoreType.DMA,
        ],
    )(wT, p16T, sidx.reshape(NTOK), zrow)

    out = pl.pallas_call(
        _final_body,
        grid=(M,),
        in_specs=[
            pl.BlockSpec((E, G), lambda i: (0, i)),
            pl.BlockSpec((16, G), lambda i: (0, i)),
            pl.BlockSpec((16, G), lambda i: (0, 0)),
            pl.BlockSpec((E, G), lambda i: (0, 0)),
            pl.BlockSpec((E, E), lambda i: (0, 0)),
            pl.BlockSpec((16, E), lambda i: (0, 0)),
        ],
        out_specs=pl.BlockSpec((G, E), lambda i: (i, 0)),
        out_shape=jax.ShapeDtypeStruct((M * G, E), F32),
    )(numT, denT, ps16T, vgT, Wo, st16)

    z_grid = out.reshape(M, P0, P1, E)

    axes = [jnp.linspace(r[0], r[1], p, dtype=F32)
            for r, p in zip(((0.0, 1.0), (0.0, 1.0)), (P0, P1))]
    grid_pts = jnp.stack(jnp.meshgrid(*axes, indexing="ij"), axis=-1)
    x_grid = jnp.broadcast_to(grid_pts[None], (M, P0, P1, 2))
    return x_grid, z_grid


# speedup vs baseline: 4.6988x; 1.0510x over previous
import jax
import jax.numpy as jnp
import numpy as np
from jax import lax
from jax.experimental import pallas as pl
from jax.experimental.pallas import tpu as pltpu
from jax.experimental.pallas import tpu_sc as plsc

E = 256
H = 8
HD = E // H
P0, P1 = 32, 32
G = P0 * P1
M = 4
N = 8192
NTOK = M * N
NC, NS = 2, 16
NW = NC * NS
CH = 128
CHS = 512
RPS = NTOK // NC
RWS = 2 * G
F32 = jnp.float32
SCALE = np.float32(1.0 / np.sqrt(HD))


def _cells_body(x0_ref, x1_ref, cell_ref, sidx_ref):
    sp = np.float32(1.0) / np.float32(P0 - 1)
    half = sp / np.float32(2.0)
    i0 = jnp.clip(jnp.floor((x0_ref[...] + half) / sp), 0.0, np.float32(P0 - 1))
    i1 = jnp.clip(jnp.floor((x1_ref[...] + half) / sp), 0.0, np.float32(P1 - 1))
    cell = (i0 * np.float32(P1) + i1).astype(jnp.int32)
    cell_ref[...] = cell
    b = lax.broadcasted_iota(jnp.int32, (M, N), 0)
    sidx_ref[...] = cell + (b % 2) * G


def _lat_body(latfT_ref, wq_ref, wk_ref, wv_ref, s_ref, e816_ref,
              qlat_ref, vgT_ref, ps16T_ref):
    latfT = latfT_ref[...]
    q = lax.dot_general(latfT, wq_ref[...], (((0,), (0,)), ((), ())),
                        preferred_element_type=F32)
    kg = lax.dot_general(latfT, wk_ref[...], (((0,), (0,)), ((), ())),
                         preferred_element_type=F32)
    vgT_ref[...] = lax.dot_general(wv_ref[...], latfT, (((0,), (0,)), ((), ())),
                                   preferred_element_type=F32)
    qlat_ref[...] = q
    l8 = jnp.dot(q * kg, s_ref[...], preferred_element_type=F32) * SCALE
    psT = jnp.exp(l8).T
    ps16T_ref[...] = lax.dot_general(e816_ref[...], psT, (((0,), (0,)), ((), ())),
                                     preferred_element_type=F32)


def _fused_w_body(z_ref, qt_ref, wk_ref, wv_ref, s_ref, e816_ref,
                  wT_ref, p16T_ref):
    zb = z_ref[...]
    k = jnp.dot(zb, wk_ref[...], preferred_element_type=F32)
    l8 = jnp.dot(qt_ref[...] * k, s_ref[...], preferred_element_type=F32) * SCALE
    pT = jnp.exp(l8).T
    vT = lax.dot_general(wv_ref[...], zb, (((0,), (1,)), ((), ())),
                         preferred_element_type=F32)
    prepT = jnp.dot(s_ref[...], pT, preferred_element_type=F32)
    wT_ref[...] = prepT * vT
    p16T_ref[...] = lax.dot_general(e816_ref[...], pT, (((0,), (0,)), ((), ())),
                                    preferred_element_type=F32)


def _final_body(numT_ref, denT_ref, ps16T_ref, vgT_ref, wo_ref, st16_ref,
                out_ref):
    ps_repT = lax.dot_general(st16_ref[...], ps16T_ref[...],
                              (((0,), (0,)), ((), ())),
                              preferred_element_type=F32)
    numerT = numT_ref[...] + ps_repT * vgT_ref[...]
    den_repT = lax.dot_general(st16_ref[...], denT_ref[...] + ps16T_ref[...],
                               (((0,), (0,)), ((), ())),
                               preferred_element_type=F32)
    attnT = numerT / den_repT
    out_ref[...] = lax.dot_general(attnT, wo_ref[...], (((0,), (0,)), ((), ())),
                                   preferred_element_type=F32)


def _sc_gather_body(qlat_hbm, cell_hbm, out_hbm,
                    idx_v0, idx_v1, rows_v0, rows_v1,
                    sem_i0, sem_i1, sem_g, sem_o0, sem_o1):
    c = lax.axis_index("c")
    s = lax.axis_index("s")
    wid = s * NC + c
    per_w = NTOK // NW
    nchunks = per_w // CH
    bufs = ((idx_v0, rows_v0, sem_i0, sem_o0), (idx_v1, rows_v1, sem_i1, sem_o1))

    def base_of(ci):
        return wid * per_w + ci * CH

    pltpu.async_copy(cell_hbm.at[pl.ds(base_of(0), CH)], idx_v0, sem_i0)

    def pair(i, carry):
        for b in range(2):
            ci = i * 2 + b
            idx_v, rows_v, sem_i, sem_o = bufs[b]
            nidx_v, _, nsem_i, _ = bufs[1 - b]
            pltpu.make_async_copy(cell_hbm.at[pl.ds(0, CH)], idx_v, sem_i).wait()

            @pl.when(ci + 1 < nchunks)
            def _():
                pltpu.async_copy(cell_hbm.at[pl.ds(base_of(ci + 1), CH)],
                                 nidx_v, nsem_i)

            @pl.when(ci >= 2)
            def _():
                pltpu.make_async_copy(rows_v, out_hbm.at[pl.ds(0, CH)],
                                      sem_o).wait()

            pltpu.async_copy(qlat_hbm.at[idx_v], rows_v, sem_g).wait()
            pltpu.async_copy(rows_v, out_hbm.at[pl.ds(base_of(ci), CH)], sem_o)
        return carry

    lax.fori_loop(0, nchunks // 2, pair, 0)
    pltpu.make_async_copy(rows_v0, out_hbm.at[pl.ds(0, CH)], sem_o0).wait()
    pltpu.make_async_copy(rows_v1, out_hbm.at[pl.ds(0, CH)], sem_o1).wait()


def _sc_scatter_body(wT_hbm, p16T_hbm, sidx_hbm, zrow_hbm,
                     numT_hbm, denT_hbm,
                     cell_v0, cell_v1, w_v0, w_v1, acc, pbuf, sh_p,
                     sem0, sem1):
    c = lax.axis_index("c")
    s = lax.axis_index("s")
    bufs = ((cell_v0, w_v0, sem0), (cell_v1, w_v1, sem1))

    def accumulate(cell_v, w_v):
        def tblk(t, carry):
            rows = cell_v[pl.ds(t * 16, 16)]
            for j in range(16):
                vals = w_v[j, pl.ds(t * 16, 16)]
                plsc.addupdate_scatter(acc, [jnp.full((16,), j, jnp.int32), rows],
                                       vals)
            return carry

        lax.fori_loop(0, CHS // 16, tblk, 0)

    def start(i, src_hbm, row0, nrows, tok0, b):
        cell_v, w_v, sem = bufs[b]
        base = tok0 + i * CHS
        cp1 = pltpu.async_copy(sidx_hbm.at[pl.ds(base, CHS)], cell_v, sem)
        cp2 = pltpu.async_copy(src_hbm.at[pl.ds(row0, nrows), pl.ds(base, CHS)],
                               w_v.at[pl.ds(0, nrows)], sem)
        return cp1, cp2

    def run_pass(src_hbm, row0, nrows, tok0, nchunks):
        start(0, src_hbm, row0, nrows, tok0, 0)

        def pair(i, carry):
            for b in range(2):
                ci = i * 2 + b
                cell_v, w_v, sem = bufs[b]
                pltpu.make_async_copy(sidx_hbm.at[pl.ds(0, CHS)], cell_v, sem).wait()
                pltpu.make_async_copy(
                    src_hbm.at[pl.ds(row0, nrows), pl.ds(0, CHS)],
                    w_v.at[pl.ds(0, nrows)], sem).wait()

                @pl.when(ci + 1 < nchunks)
                def _():
                    start(ci + 1, src_hbm, row0, nrows, tok0, 1 - b)

                accumulate(cell_v, w_v)
            return carry

        lax.fori_loop(0, nchunks // 2, pair, 0)

    pltpu.sync_copy(zrow_hbm, acc)
    run_pass(wT_hbm, s * 16, 16, c * RPS, RPS // CHS)
    pltpu.sync_copy(acc, numT_hbm.at[pl.ds(s * 16, 16), pl.ds(c * RWS, RWS)])

    pltpu.sync_copy(zrow_hbm, acc)
    per_tile = RPS // NS
    run_pass(p16T_hbm, 0, 16, c * RPS + s * per_tile, per_tile // CHS)
    pltpu.sync_copy(acc, sh_p.at[s])
    plsc.subcore_barrier()

    cs = RWS // NS
    pltpu.sync_copy(sh_p.at[:, :, pl.ds(s * cs, cs)], pbuf)

    def comb(i, carry):
        h = i // (cs // 16)
        c0 = (i % (cs // 16)) * 16

        def part(j, a):
            return a + pbuf[j, h, pl.ds(c0, 16)]

        w_v0[h, pl.ds(c0, 16)] = lax.fori_loop(0, NS, part, jnp.zeros((16,), F32))
        return carry

    lax.fori_loop(0, 16 * (cs // 16), comb, 0)
    pltpu.sync_copy(w_v0.at[:, pl.ds(0, cs)],
                    denT_hbm.at[:, pl.ds(c * RWS + s * cs, cs)])


def kernel(x, z, latents, Wq, Wk, Wv, Wo):
    hid = np.arange(E) // HD
    s_mat = jnp.asarray((hid[:, None] == np.arange(H)[None, :]).astype(np.float32))
    e816 = jnp.asarray(np.eye(H, 16, dtype=np.float32))
    st16 = jnp.asarray((hid[None, :] == np.arange(16)[:, None]).astype(np.float32))

    cell, sidx = pl.pallas_call(
        _cells_body,
        out_shape=(jax.ShapeDtypeStruct((M, N), jnp.int32),
                   jax.ShapeDtypeStruct((M, N), jnp.int32)),
    )(x[..., 0], x[..., 1])

    latfT = latents.reshape(G, E).T
    qlat, vgT, ps16T = pl.pallas_call(
        _lat_body,
        out_shape=(jax.ShapeDtypeStruct((G, E), F32),
                   jax.ShapeDtypeStruct((E, G), F32),
                   jax.ShapeDtypeStruct((16, G), F32)),
    )(latfT, Wq, Wk, Wv, s_mat, e816)

    mesh = plsc.VectorSubcoreMesh(core_axis_name="c", subcore_axis_name="s",
                                  num_cores=NC, num_subcores=NS)

    qt = pl.kernel(
        _sc_gather_body,
        out_type=jax.ShapeDtypeStruct((NTOK, E), F32),
        mesh=mesh,
        scratch_types=[
            pltpu.VMEM((CH,), jnp.int32),
            pltpu.VMEM((CH,), jnp.int32),
            pltpu.VMEM((CH, E), F32),
            pltpu.VMEM((CH, E), F32),
            pltpu.SemaphoreType.DMA,
            pltpu.SemaphoreType.DMA,
            pltpu.SemaphoreType.DMA,
            pltpu.SemaphoreType.DMA,
            pltpu.SemaphoreType.DMA,
        ],
    )(qlat, cell.reshape(NTOK))

    BT = 1024
    wT, p16T = pl.pallas_call(
        _fused_w_body,
        grid=(NTOK // BT,),
        in_specs=[
            pl.BlockSpec((BT, E), lambda i: (i, 0)),
            pl.BlockSpec((BT, E), lambda i: (i, 0)),
            pl.BlockSpec((E, E), lambda i: (0, 0)),
            pl.BlockSpec((E, E), lambda i: (0, 0)),
            pl.BlockSpec((E, H), lambda i: (0, 0)),
            pl.BlockSpec((H, 16), lambda i: (0, 0)),
        ],
        out_specs=(pl.BlockSpec((E, BT), lambda i: (0, i)),
                   pl.BlockSpec((16, BT), lambda i: (0, i))),
        out_shape=(jax.ShapeDtypeStruct((E, NTOK), F32),
                   jax.ShapeDtypeStruct((16, NTOK), F32)),
    )(z.reshape(NTOK, E), qt, Wk, Wv, s_mat, e816)

    zrow = jnp.zeros((16, RWS), F32)
    numT, denT = pl.kernel(
        _sc_scatter_body,
        out_type=(jax.ShapeDtypeStruct((E, M * G), F32),
                  jax.ShapeDtypeStruct((16, M * G), F32)),
        mesh=mesh,
        compiler_params=pltpu.CompilerParams(needs_layout_passes=False),
        scratch_types=[
            pltpu.VMEM((CHS,), jnp.int32),
            pltpu.VMEM((CHS,), jnp.int32),
            pltpu.VMEM((16, CHS), F32),
            pltpu.VMEM((16, CHS), F32),
            pltpu.VMEM((16, RWS), F32),
            pltpu.VMEM((NS, 16, RWS // NS), F32),
            pltpu.VMEM_SHARED((NS, 16, RWS), F32),
            pltpu.SemaphoreType.DMA,
            pltpu.SemaphoreType.DMA,
        ],
    )(wT, p16T, sidx.reshape(NTOK), zrow)

    out = pl.pallas_call(
        _final_body,
        grid=(M,),
        in_specs=[
            pl.BlockSpec((E, G), lambda i: (0, i)),
            pl.BlockSpec((16, G), lambda i: (0, i)),
            pl.BlockSpec((16, G), lambda i: (0, 0)),
            pl.BlockSpec((E, G), lambda i: (0, 0)),
            pl.BlockSpec((E, E), lambda i: (0, 0)),
            pl.BlockSpec((16, E), lambda i: (0, 0)),
        ],
        out_specs=pl.BlockSpec((G, E), lambda i: (i, 0)),
        out_shape=jax.ShapeDtypeStruct((M * G, E), F32),
    )(numT, denT, ps16T, vgT, Wo, st16)

    z_grid = out.reshape(M, P0, P1, E)

    axes = [jnp.linspace(r[0], r[1], p, dtype=F32)
            for r, p in zip(((0.0, 1.0), (0.0, 1.0)), (P0, P1))]
    grid_pts = jnp.stack(jnp.meshgrid(*axes, indexing="ij"), axis=-1)
    x_grid = jnp.broadcast_to(grid_pts[None], (M, P0, P1, 2))
    return x_grid, z_grid


# speedup vs baseline: 6.8055x; 1.4484x over previous
import functools

import jax
import jax.numpy as jnp
import numpy as np
from jax import lax
from jax.experimental import pallas as pl
from jax.experimental.pallas import tpu as pltpu
from jax.experimental.pallas import tpu_sc as plsc

E = 256
H = 8
HD = E // H
P0, P1 = 32, 32
G = P0 * P1
M = 4
N = 8192
NTOK = M * N
HTOK = NTOK // 2
NC, NS = 2, 16
NW = NC * NS
CH = 128
CHS = 512
RPS = HTOK // NC
F32 = jnp.float32
SCALE = np.float32(1.0 / np.sqrt(HD))


def _lat_body(latfT_ref, x0_ref, x1_ref, wq_ref, wk_ref, wv_ref, s_ref,
              e816_ref, qlat_ref, vgT_ref, ps16T_ref, cell_ref):
    sp = np.float32(1.0) / np.float32(P0 - 1)
    half = sp / np.float32(2.0)
    i0 = jnp.clip(jnp.floor((x0_ref[...] + half) / sp), 0.0, np.float32(P0 - 1))
    i1 = jnp.clip(jnp.floor((x1_ref[...] + half) / sp), 0.0, np.float32(P1 - 1))
    cell_ref[...] = (i0 * np.float32(P1) + i1).astype(jnp.int32)

    latfT = latfT_ref[...]
    q = lax.dot_general(latfT, wq_ref[...], (((0,), (0,)), ((), ())),
                        preferred_element_type=F32)
    kg = lax.dot_general(latfT, wk_ref[...], (((0,), (0,)), ((), ())),
                         preferred_element_type=F32)
    vgT_ref[...] = lax.dot_general(wv_ref[...], latfT, (((0,), (0,)), ((), ())),
                                   preferred_element_type=F32)
    qbits = lax.bitcast_convert_type(q, jnp.int32)
    qr = (qbits + 0x7FFF + ((qbits >> 16) & 1)) >> 16
    lo = qr[:, :E // 2] & 0xFFFF
    hi = qr[:, E // 2:] << 16
    qlat_ref[...] = lo | hi
    l8 = jnp.dot(q * kg, s_ref[...], preferred_element_type=F32) * SCALE
    psT = jnp.exp(l8).T
    ps16T_ref[...] = lax.dot_general(e816_ref[...], psT, (((0,), (0,)), ((), ())),
                                     preferred_element_type=F32)


def _fused_w_body(z_ref, qt_ref, wk_ref, wv_ref, s_ref, e816_ref,
                  wT_ref, p16T_ref):
    zb = z_ref[...]
    k = jnp.dot(zb, wk_ref[...], preferred_element_type=F32)
    packed = qt_ref[...]
    qa = lax.bitcast_convert_type(packed << 16, F32)
    qb = lax.bitcast_convert_type(packed & np.int32(-65536), F32)
    qk = jnp.concatenate([qa * k[:, :E // 2], qb * k[:, E // 2:]], axis=1)
    l8 = jnp.dot(qk, s_ref[...], preferred_element_type=F32) * SCALE
    pT = jnp.exp(l8).T
    vT = lax.dot_general(wv_ref[...], zb, (((0,), (1,)), ((), ())),
                         preferred_element_type=F32)
    prepT = jnp.dot(s_ref[...], pT, preferred_element_type=F32)
    wT_ref[...] = prepT * vT
    p16T_ref[...] = lax.dot_general(e816_ref[...], pT, (((0,), (0,)), ((), ())),
                                    preferred_element_type=F32)


def _final_body(numT_ref, denT_ref, ps16T_ref, vgT_ref, wo_ref, st16_ref,
                out_ref):
    ps_repT = lax.dot_general(st16_ref[...], ps16T_ref[...],
                              (((0,), (0,)), ((), ())),
                              preferred_element_type=F32)
    numerT = numT_ref[...] + ps_repT * vgT_ref[...]
    den_repT = lax.dot_general(st16_ref[...], denT_ref[...] + ps16T_ref[...],
                               (((0,), (0,)), ((), ())),
                               preferred_element_type=F32)
    attnT = numerT / den_repT
    out_ref[...] = lax.dot_general(attnT, wo_ref[...], (((0,), (0,)), ((), ())),
                                   preferred_element_type=F32)


def _mk_gather_body(tok0):
    def body(qlat_hbm, cell_hbm, out_hbm,
             idx_v0, idx_v1, rows_v0, rows_v1, stage_v, sh_tbl,
             sem_i0, sem_i1, sem_g, sem_o0, sem_o1):
        c = lax.axis_index("c")
        s = lax.axis_index("s")
        wid = s * NC + c
        per_w = HTOK // NW
        nchunks = per_w // CH
        bufs = ((idx_v0, rows_v0, sem_i0, sem_o0),
                (idx_v1, rows_v1, sem_i1, sem_o1))

        def base_of(ci):
            return wid * per_w + ci * CH

        pltpu.async_copy(cell_hbm.at[pl.ds(tok0 + base_of(0), CH)], idx_v0,
                         sem_i0)

        rs = G // NS
        pltpu.sync_copy(qlat_hbm.at[pl.ds(s * rs, rs)], stage_v)
        pltpu.sync_copy(stage_v, sh_tbl.at[pl.ds(s * rs, rs)])
        plsc.subcore_barrier()

        def pair(i, carry):
            for b in range(2):
                ci = i * 2 + b
                idx_v, rows_v, sem_i, sem_o = bufs[b]
                nidx_v, _, nsem_i, _ = bufs[1 - b]
                pltpu.make_async_copy(cell_hbm.at[pl.ds(0, CH)], idx_v,
                                      sem_i).wait()

                @pl.when(ci + 1 < nchunks)
                def _():
                    pltpu.async_copy(
                        cell_hbm.at[pl.ds(tok0 + base_of(ci + 1), CH)],
                        nidx_v, nsem_i)

                @pl.when(ci >= 2)
                def _():
                    pltpu.make_async_copy(rows_v, out_hbm.at[pl.ds(0, CH)],
                                          sem_o).wait()

                pltpu.async_copy(sh_tbl.at[idx_v], rows_v, sem_g).wait()
                pltpu.async_copy(rows_v, out_hbm.at[pl.ds(base_of(ci), CH)],
                                 sem_o)
            return carry

        lax.fori_loop(0, nchunks // 2, pair, 0)
        pltpu.make_async_copy(rows_v0, out_hbm.at[pl.ds(0, CH)], sem_o0).wait()
        pltpu.make_async_copy(rows_v1, out_hbm.at[pl.ds(0, CH)], sem_o1).wait()

    return body


def _mk_scatter_body(tok0):
    def body(wT_hbm, p16T_hbm, cell_hbm, numT_hbm, denT_hbm,
             cell_v0, cell_v1, w_v0, w_v1, acc, pbuf, sh_p, sem0, sem1):
        c = lax.axis_index("c")
        s = lax.axis_index("s")
        bufs = ((cell_v0, w_v0, sem0), (cell_v1, w_v1, sem1))
        z16 = jnp.zeros((16,), F32)

        def zero_acc():
            def zr(i, carry):
                acc[i // (G // 16), pl.ds((i % (G // 16)) * 16, 16)] = z16
                return carry

            lax.fori_loop(0, 16 * (G // 16), zr, 0)

        def accumulate(cell_v, w_v):
            def tblk(t, carry):
                rows = cell_v[pl.ds(t * 16, 16)]
                vals = [w_v[j, pl.ds(t * 16, 16)] for j in range(16)]
                for j in range(16):
                    plsc.addupdate_scatter(
                        acc, [jnp.full((16,), j, jnp.int32), rows], vals[j])
                return carry

            lax.fori_loop(0, CHS // 16, tblk, 0)

        def start(i, src_hbm, row0, sc_tok0, b):
            cell_v, w_v, sem = bufs[b]
            base = sc_tok0 + i * CHS
            pltpu.async_copy(cell_hbm.at[pl.ds(base, CHS)], cell_v, sem)
            pltpu.async_copy(src_hbm.at[pl.ds(row0, 16),
                                        pl.ds(base - tok0, CHS)], w_v, sem)

        def wait_bufs(b, src_hbm, row0):
            cell_v, w_v, sem = bufs[b]
            pltpu.make_async_copy(cell_hbm.at[pl.ds(0, CHS)], cell_v,
                                  sem).wait()
            pltpu.make_async_copy(src_hbm.at[pl.ds(row0, 16), pl.ds(0, CHS)],
                                  w_v, sem).wait()

        def run_pass(src_hbm, row0, sc_tok0, nchunks):
            start(0, src_hbm, row0, sc_tok0, 0)

            def pair(i, carry):
                for b in range(2):
                    ci = i * 2 + b
                    wait_bufs(b, src_hbm, row0)

                    @pl.when(ci + 1 < nchunks)
                    def _():
                        start(ci + 1, src_hbm, row0, sc_tok0, 1 - b)

                    accumulate(bufs[b][0], bufs[b][1])
                return carry

            lax.fori_loop(0, nchunks // 2, pair, 0)

        zero_acc()
        run_pass(wT_hbm, s * 16, tok0 + c * RPS, RPS // CHS)
        pltpu.sync_copy(acc, numT_hbm.at[pl.ds(s * 16, 16), pl.ds(c * G, G)])

        zero_acc()
        per_tile = RPS // NS
        start(0, p16T_hbm, 0, tok0 + c * RPS + s * per_tile, 0)
        wait_bufs(0, p16T_hbm, 0)
        accumulate(cell_v0, w_v0)
        pltpu.sync_copy(acc, sh_p.at[s])
        plsc.subcore_barrier()

        cs = 128
        @pl.when(s < 8)
        def _():
            pltpu.sync_copy(sh_p.at[:, :, pl.ds(s * cs, cs)], pbuf)

            def comb(i, carry):
                h = i // (cs // 16)
                c0 = (i % (cs // 16)) * 16

                def part(j, a):
                    return a + pbuf[j, h, pl.ds(c0, 16)]

                w_v0[h, pl.ds(c0, 16)] = lax.fori_loop(0, NS, part,
                                                       jnp.zeros((16,), F32))
                return carry

            lax.fori_loop(0, 16 * (cs // 16), comb, 0)
            pltpu.sync_copy(w_v0.at[:, pl.ds(0, cs)],
                            denT_hbm.at[:, pl.ds(c * G + s * cs, cs)])

    return body


def kernel(x, z, latents, Wq, Wk, Wv, Wo):
    hid = np.arange(E) // HD
    s_mat = jnp.asarray((hid[:, None] == np.arange(H)[None, :]).astype(np.float32))
    e816 = jnp.asarray(np.eye(H, 16, dtype=np.float32))
    st16 = jnp.asarray((hid[None, :] == np.arange(16)[:, None]).astype(np.float32))

    latfT = latents.reshape(G, E).T
    qlat, vgT, ps16T, cell = pl.pallas_call(
        _lat_body,
        out_shape=(jax.ShapeDtypeStruct((G, E // 2), jnp.int32),
                   jax.ShapeDtypeStruct((E, G), F32),
                   jax.ShapeDtypeStruct((16, G), F32),
                   jax.ShapeDtypeStruct((M, N), jnp.int32)),
    )(latfT, x[..., 0], x[..., 1], Wq, Wk, Wv, s_mat, e816)
    cell_flat = cell.reshape(NTOK)

    mesh = plsc.VectorSubcoreMesh(core_axis_name="c", subcore_axis_name="s",
                                  num_cores=NC, num_subcores=NS)
    sc_params = pltpu.CompilerParams(needs_layout_passes=False)

    def gather_half(h):
        return pl.kernel(
            _mk_gather_body(h * HTOK),
            out_type=jax.ShapeDtypeStruct((HTOK, E // 2), jnp.int32),
            mesh=mesh,
            scratch_types=[
                pltpu.VMEM((CH,), jnp.int32),
                pltpu.VMEM((CH,), jnp.int32),
                pltpu.VMEM((CH, E // 2), jnp.int32),
                pltpu.VMEM((CH, E // 2), jnp.int32),
                pltpu.VMEM((G // NS, E // 2), jnp.int32),
                pltpu.VMEM_SHARED((G, E // 2), jnp.int32),
                pltpu.SemaphoreType.DMA,
                pltpu.SemaphoreType.DMA,
                pltpu.SemaphoreType.DMA,
                pltpu.SemaphoreType.DMA,
                pltpu.SemaphoreType.DMA,
            ],
        )(qlat, cell_flat)

    BT = 1024

    def fused_half(h):
        nblk = HTOK // BT
        return pl.pallas_call(
            _fused_w_body,
            grid=(nblk,),
            in_specs=[
                pl.BlockSpec((BT, E), lambda i: (i + h * nblk, 0)),
                pl.BlockSpec((BT, E // 2), lambda i: (i, 0)),
                pl.BlockSpec((E, E), lambda i: (0, 0)),
                pl.BlockSpec((E, E), lambda i: (0, 0)),
                pl.BlockSpec((E, H), lambda i: (0, 0)),
                pl.BlockSpec((H, 16), lambda i: (0, 0)),
            ],
            out_specs=(pl.BlockSpec((E, BT), lambda i: (0, i)),
                       pl.BlockSpec((16, BT), lambda i: (0, i))),
            out_shape=(jax.ShapeDtypeStruct((E, HTOK), F32),
                       jax.ShapeDtypeStruct((16, HTOK), F32)),
        )(z.reshape(NTOK, E), qt[h], Wk, Wv, s_mat, e816)

    def scatter_half(h):
        return pl.kernel(
            _mk_scatter_body(h * HTOK),
            out_type=(jax.ShapeDtypeStruct((E, 2 * G), F32),
                      jax.ShapeDtypeStruct((16, 2 * G), F32)),
            mesh=mesh,
            compiler_params=sc_params,
            scratch_types=[
                pltpu.VMEM((CHS,), jnp.int32),
                pltpu.VMEM((CHS,), jnp.int32),
                pltpu.VMEM((16, CHS), F32),
                pltpu.VMEM((16, CHS), F32),
                pltpu.VMEM((16, G), F32),
                pltpu.VMEM((NS, 16, 128), F32),
                pltpu.VMEM_SHARED((NS, 16, G), F32),
                pltpu.SemaphoreType.DMA,
                pltpu.SemaphoreType.DMA,
            ],
        )(wT[h], p16T[h], cell_flat)

    def final_half(h):
        return pl.pallas_call(
            _final_body,
            grid=(2,),
            in_specs=[
                pl.BlockSpec((E, G), lambda i: (0, i)),
                pl.BlockSpec((16, G), lambda i: (0, i)),
                pl.BlockSpec((16, G), lambda i: (0, 0)),
                pl.BlockSpec((E, G), lambda i: (0, 0)),
                pl.BlockSpec((E, E), lambda i: (0, 0)),
                pl.BlockSpec((16, E), lambda i: (0, 0)),
            ],
            out_specs=pl.BlockSpec((G, E), lambda i: (i, 0)),
            out_shape=jax.ShapeDtypeStruct((2 * G, E), F32),
        )(numT[h], denT[h], ps16T, vgT, Wo, st16)

    qt, wT, p16T, numT, denT, out = {}, {}, {}, {}, {}, {}
    for h in range(2):
        qt[h] = gather_half(h)
        wT[h], p16T[h] = fused_half(h)
        numT[h], denT[h] = scatter_half(h)
        out[h] = final_half(h)

    z_grid = jnp.concatenate([out[0], out[1]], axis=0).reshape(M, P0, P1, E)

    axes = [jnp.linspace(r[0], r[1], p, dtype=F32)
            for r, p in zip(((0.0, 1.0), (0.0, 1.0)), (P0, P1))]
    grid_pts = jnp.stack(jnp.meshgrid(*axes, indexing="ij"), axis=-1)
    x_grid = jnp.broadcast_to(grid_pts[None], (M, P0, P1, 2))
    return x_grid, z_grid


# speedup vs baseline: 7.0272x; 1.0326x over previous
import functools

import jax
import jax.numpy as jnp
import numpy as np
from jax import lax
from jax.experimental import pallas as pl
from jax.experimental.pallas import tpu as pltpu
from jax.experimental.pallas import tpu_sc as plsc

E = 256
H = 8
HD = E // H
P0, P1 = 32, 32
G = P0 * P1
M = 4
N = 8192
NTOK = M * N
HTOK = NTOK // 2
NC, NS = 2, 16
NW = NC * NS
CH = 128
CHS = 1024
RPS = HTOK // NC
F32 = jnp.float32
SCALE = np.float32(1.0 / np.sqrt(HD))


def _lat_body(latfT_ref, x0_ref, x1_ref, wq_ref, wk_ref, wv_ref, s_ref,
              e816_ref, qlat_ref, vgT_ref, ps16T_ref, cell_ref):
    sp = np.float32(1.0) / np.float32(P0 - 1)
    half = sp / np.float32(2.0)
    i0 = jnp.clip(jnp.floor((x0_ref[...] + half) / sp), 0.0, np.float32(P0 - 1))
    i1 = jnp.clip(jnp.floor((x1_ref[...] + half) / sp), 0.0, np.float32(P1 - 1))
    cell_ref[...] = (i0 * np.float32(P1) + i1).astype(jnp.int32)

    latfT = latfT_ref[...]
    q = lax.dot_general(latfT, wq_ref[...], (((0,), (0,)), ((), ())),
                        preferred_element_type=F32)
    kg = lax.dot_general(latfT, wk_ref[...], (((0,), (0,)), ((), ())),
                         preferred_element_type=F32)
    vgT_ref[...] = lax.dot_general(wv_ref[...], latfT, (((0,), (0,)), ((), ())),
                                   preferred_element_type=F32)
    qbits = lax.bitcast_convert_type(q, jnp.int32)
    qr = (qbits + 0x7FFF + ((qbits >> 16) & 1)) >> 16
    lo = qr[:, :E // 2] & 0xFFFF
    hi = qr[:, E // 2:] << 16
    qlat_ref[...] = lo | hi
    l8 = jnp.dot(q * kg, s_ref[...], preferred_element_type=F32) * SCALE
    psT = jnp.exp(l8).T
    ps16T_ref[...] = lax.dot_general(e816_ref[...], psT, (((0,), (0,)), ((), ())),
                                     preferred_element_type=F32)


def _fused_w_body(z_ref, qt_ref, wk_ref, wv_ref, s_ref, e816_ref,
                  wT_ref, p16T_ref):
    zb = z_ref[...]
    k = jnp.dot(zb, wk_ref[...], preferred_element_type=F32)
    packed = qt_ref[...]
    qa = lax.bitcast_convert_type(packed << 16, F32)
    qb = lax.bitcast_convert_type(packed & np.int32(-65536), F32)
    qk = jnp.concatenate([qa * k[:, :E // 2], qb * k[:, E // 2:]], axis=1)
    l8 = jnp.dot(qk, s_ref[...], preferred_element_type=F32) * SCALE
    pT = jnp.exp(l8).T
    vT = lax.dot_general(wv_ref[...], zb, (((0,), (1,)), ((), ())),
                         preferred_element_type=F32)
    prepT = jnp.dot(s_ref[...], pT, preferred_element_type=F32)
    wT_ref[...] = prepT * vT
    p16T_ref[...] = lax.dot_general(e816_ref[...], pT, (((0,), (0,)), ((), ())),
                                    preferred_element_type=F32)


def _final_body(numT_ref, denT_ref, ps16T_ref, vgT_ref, wo_ref, st16_ref,
                out_ref):
    ps_repT = lax.dot_general(st16_ref[...], ps16T_ref[...],
                              (((0,), (0,)), ((), ())),
                              preferred_element_type=F32)
    numerT = numT_ref[...] + ps_repT * vgT_ref[...]
    den_repT = lax.dot_general(st16_ref[...], denT_ref[...] + ps16T_ref[...],
                               (((0,), (0,)), ((), ())),
                               preferred_element_type=F32)
    attnT = numerT / den_repT
    out_ref[...] = lax.dot_general(attnT, wo_ref[...], (((0,), (0,)), ((), ())),
                                   preferred_element_type=F32)


def _mk_gather_body(tok0):
    def body(qlat_hbm, cell_hbm, out_hbm,
             idx_v0, idx_v1, rows_v0, rows_v1, stage_v, sh_tbl,
             sem_i0, sem_i1, sem_g, sem_o0, sem_o1):
        c = lax.axis_index("c")
        s = lax.axis_index("s")
        wid = s * NC + c
        per_w = HTOK // NW
        nchunks = per_w // CH
        bufs = ((idx_v0, rows_v0, sem_i0, sem_o0),
                (idx_v1, rows_v1, sem_i1, sem_o1))

        def base_of(ci):
            return wid * per_w + ci * CH

        pltpu.async_copy(cell_hbm.at[pl.ds(tok0 + base_of(0), CH)], idx_v0,
                         sem_i0)

        rs = G // NS
        pltpu.sync_copy(qlat_hbm.at[pl.ds(s * rs, rs)], stage_v)
        pltpu.sync_copy(stage_v, sh_tbl.at[pl.ds(s * rs, rs)])
        plsc.subcore_barrier()

        def pair(i, carry):
            for b in range(2):
                ci = i * 2 + b
                idx_v, rows_v, sem_i, sem_o = bufs[b]
                nidx_v, _, nsem_i, _ = bufs[1 - b]
                pltpu.make_async_copy(cell_hbm.at[pl.ds(0, CH)], idx_v,
                                      sem_i).wait()

                @pl.when(ci + 1 < nchunks)
                def _():
                    pltpu.async_copy(
                        cell_hbm.at[pl.ds(tok0 + base_of(ci + 1), CH)],
                        nidx_v, nsem_i)

                @pl.when(ci >= 2)
                def _():
                    pltpu.make_async_copy(rows_v, out_hbm.at[pl.ds(0, CH)],
                                          sem_o).wait()

                pltpu.async_copy(sh_tbl.at[idx_v], rows_v, sem_g).wait()
                pltpu.async_copy(rows_v, out_hbm.at[pl.ds(base_of(ci), CH)],
                                 sem_o)
            return carry

        lax.fori_loop(0, nchunks // 2, pair, 0)
        pltpu.make_async_copy(rows_v0, out_hbm.at[pl.ds(0, CH)], sem_o0).wait()
        pltpu.make_async_copy(rows_v1, out_hbm.at[pl.ds(0, CH)], sem_o1).wait()

    return body


def _mk_scatter_body(tok0):
    def body(wT_hbm, p16T_hbm, cell_hbm, numT_hbm, denT_hbm,
             cell_v0, cell_v1, w_v0, w_v1, acc, pbuf, sh_p, sem0, sem1):
        c = lax.axis_index("c")
        s = lax.axis_index("s")
        bufs = ((cell_v0, w_v0, sem0), (cell_v1, w_v1, sem1))
        z16 = jnp.zeros((16,), F32)

        def zero_acc():
            def zr(i, carry):
                acc[i // (G // 16), pl.ds((i % (G // 16)) * 16, 16)] = z16
                return carry

            lax.fori_loop(0, 16 * (G // 16), zr, 0)

        def accumulate(cell_v, w_v, nt):
            def tblk(t, carry):
                rows = cell_v[pl.ds(t * 16, 16)]
                vals = [w_v[j, pl.ds(t * 16, 16)] for j in range(16)]
                for j in range(16):
                    plsc.addupdate_scatter(
                        acc, [jnp.full((16,), j, jnp.int32), rows], vals[j])
                return carry

            lax.fori_loop(0, nt // 16, tblk, 0)

        def start(i, src_hbm, row0, sc_tok0, b, nt):
            cell_v, w_v, sem = bufs[b]
            base = sc_tok0 + i * CHS
            pltpu.async_copy(cell_hbm.at[pl.ds(base, nt)],
                             cell_v.at[pl.ds(0, nt)], sem)
            pltpu.async_copy(src_hbm.at[pl.ds(row0, 16),
                                        pl.ds(base - tok0, nt)],
                             w_v.at[:, pl.ds(0, nt)], sem)

        def wait_bufs(b, src_hbm, row0, nt):
            cell_v, w_v, sem = bufs[b]
            pltpu.make_async_copy(cell_hbm.at[pl.ds(0, nt)],
                                  cell_v.at[pl.ds(0, nt)], sem).wait()
            pltpu.make_async_copy(src_hbm.at[pl.ds(row0, 16), pl.ds(0, nt)],
                                  w_v.at[:, pl.ds(0, nt)], sem).wait()

        def run_pass(src_hbm, row0, sc_tok0, nchunks):
            start(0, src_hbm, row0, sc_tok0, 0, CHS)

            def pair(i, carry):
                for b in range(2):
                    ci = i * 2 + b
                    wait_bufs(b, src_hbm, row0, CHS)

                    @pl.when(ci + 1 < nchunks)
                    def _():
                        start(ci + 1, src_hbm, row0, sc_tok0, 1 - b, CHS)

                    accumulate(bufs[b][0], bufs[b][1], CHS)
                return carry

            lax.fori_loop(0, nchunks // 2, pair, 0)

        zero_acc()
        run_pass(wT_hbm, s * 16, tok0 + c * RPS, RPS // CHS)
        pltpu.sync_copy(acc, numT_hbm.at[pl.ds(s * 16, 16), pl.ds(c * G, G)])

        zero_acc()
        per_tile = RPS // NS
        start(0, p16T_hbm, 0, tok0 + c * RPS + s * per_tile, 0, per_tile)
        wait_bufs(0, p16T_hbm, 0, per_tile)
        accumulate(cell_v0, w_v0, per_tile)
        pltpu.sync_copy(acc, sh_p.at[s])
        plsc.subcore_barrier()

        cs = 128
        @pl.when(s < 8)
        def _():
            pltpu.sync_copy(sh_p.at[:, :, pl.ds(s * cs, cs)], pbuf)

            def comb(i, carry):
                h = i // (cs // 16)
                c0 = (i % (cs // 16)) * 16

                def part(j, a):
                    return a + pbuf[j, h, pl.ds(c0, 16)]

                w_v0[h, pl.ds(c0, 16)] = lax.fori_loop(0, NS, part,
                                                       jnp.zeros((16,), F32))
                return carry

            lax.fori_loop(0, 16 * (cs // 16), comb, 0)
            pltpu.sync_copy(w_v0.at[:, pl.ds(0, cs)],
                            denT_hbm.at[:, pl.ds(c * G + s * cs, cs)])

    return body


def kernel(x, z, latents, Wq, Wk, Wv, Wo):
    hid = np.arange(E) // HD
    s_mat = jnp.asarray((hid[:, None] == np.arange(H)[None, :]).astype(np.float32))
    e816 = jnp.asarray(np.eye(H, 16, dtype=np.float32))
    st16 = jnp.asarray((hid[None, :] == np.arange(16)[:, None]).astype(np.float32))

    latfT = latents.reshape(G, E).T
    qlat, vgT, ps16T, cell = pl.pallas_call(
        _lat_body,
        out_shape=(jax.ShapeDtypeStruct((G, E // 2), jnp.int32),
                   jax.ShapeDtypeStruct((E, G), F32),
                   jax.ShapeDtypeStruct((16, G), F32),
                   jax.ShapeDtypeStruct((M, N), jnp.int32)),
    )(latfT, x[..., 0], x[..., 1], Wq, Wk, Wv, s_mat, e816)
    cell_flat = cell.reshape(NTOK)

    mesh = plsc.VectorSubcoreMesh(core_axis_name="c", subcore_axis_name="s",
                                  num_cores=NC, num_subcores=NS)
    sc_params = pltpu.CompilerParams(needs_layout_passes=False)

    def gather_half(h):
        return pl.kernel(
            _mk_gather_body(h * HTOK),
            out_type=jax.ShapeDtypeStruct((HTOK, E // 2), jnp.int32),
            mesh=mesh,
            scratch_types=[
                pltpu.VMEM((CH,), jnp.int32),
                pltpu.VMEM((CH,), jnp.int32),
                pltpu.VMEM((CH, E // 2), jnp.int32),
                pltpu.VMEM((CH, E // 2), jnp.int32),
                pltpu.VMEM((G // NS, E // 2), jnp.int32),
                pltpu.VMEM_SHARED((G, E // 2), jnp.int32),
                pltpu.SemaphoreType.DMA,
                pltpu.SemaphoreType.DMA,
                pltpu.SemaphoreType.DMA,
                pltpu.SemaphoreType.DMA,
                pltpu.SemaphoreType.DMA,
            ],
        )(qlat, cell_flat)

    BT = 1024

    def fused_half(h):
        nblk = HTOK // BT
        return pl.pallas_call(
            _fused_w_body,
            grid=(nblk,),
            in_specs=[
                pl.BlockSpec((BT, E), lambda i: (i + h * nblk, 0)),
                pl.BlockSpec((BT, E // 2), lambda i: (i, 0)),
                pl.BlockSpec((E, E), lambda i: (0, 0)),
                pl.BlockSpec((E, E), lambda i: (0, 0)),
                pl.BlockSpec((E, H), lambda i: (0, 0)),
                pl.BlockSpec((H, 16), lambda i: (0, 0)),
            ],
            out_specs=(pl.BlockSpec((E, BT), lambda i: (0, i)),
                       pl.BlockSpec((16, BT), lambda i: (0, i))),
            out_shape=(jax.ShapeDtypeStruct((E, HTOK), F32),
                       jax.ShapeDtypeStruct((16, HTOK), F32)),
        )(z.reshape(NTOK, E), qt[h], Wk, Wv, s_mat, e816)

    def scatter_half(h):
        return pl.kernel(
            _mk_scatter_body(h * HTOK),
            out_type=(jax.ShapeDtypeStruct((E, 2 * G), F32),
                      jax.ShapeDtypeStruct((16, 2 * G), F32)),
            mesh=mesh,
            compiler_params=sc_params,
            scratch_types=[
                pltpu.VMEM((CHS,), jnp.int32),
                pltpu.VMEM((CHS,), jnp.int32),
                pltpu.VMEM((16, CHS), F32),
                pltpu.VMEM((16, CHS), F32),
                pltpu.VMEM((16, G), F32),
                pltpu.VMEM((NS, 16, 128), F32),
                pltpu.VMEM_SHARED((NS, 16, G), F32),
                pltpu.SemaphoreType.DMA,
                pltpu.SemaphoreType.DMA,
            ],
        )(wT[h], p16T[h], cell_flat)

    def final_half(h):
        return pl.pallas_call(
            _final_body,
            grid=(2,),
            in_specs=[
                pl.BlockSpec((E, G), lambda i: (0, i)),
                pl.BlockSpec((16, G), lambda i: (0, i)),
                pl.BlockSpec((16, G), lambda i: (0, 0)),
                pl.BlockSpec((E, G), lambda i: (0, 0)),
                pl.BlockSpec((E, E), lambda i: (0, 0)),
                pl.BlockSpec((16, E), lambda i: (0, 0)),
            ],
            out_specs=pl.BlockSpec((G, E), lambda i: (i, 0)),
            out_shape=jax.ShapeDtypeStruct((2 * G, E), F32),
        )(numT[h], denT[h], ps16T, vgT, Wo, st16)

    qt, wT, p16T, numT, denT, out = {}, {}, {}, {}, {}, {}
    for h in range(2):
        qt[h] = gather_half(h)
        wT[h], p16T[h] = fused_half(h)
        numT[h], denT[h] = scatter_half(h)
        out[h] = final_half(h)

    z_grid = jnp.concatenate([out[0], out[1]], axis=0).reshape(M, P0, P1, E)

    axes = [jnp.linspace(r[0], r[1], p, dtype=F32)
            for r, p in zip(((0.0, 1.0), (0.0, 1.0)), (P0, P1))]
    grid_pts = jnp.stack(jnp.meshgrid(*axes, indexing="ij"), axis=-1)
    x_grid = jnp.broadcast_to(grid_pts[None], (M, P0, P1, 2))
    return x_grid, z_grid
